# Initial kernel scaffold; baseline (speedup 1.0000x reference)
import jax
import jax.numpy as jnp
from jax.experimental import pallas as pl


def kernel(fine_grained, coarse_pre, conv1_w, conv2_w, conv3_w, prelu_w):
    raise NotImplementedError("write your pallas kernel here")


# speedup vs baseline: 14.0188x; 14.0188x over previous
import functools

import jax
import jax.numpy as jnp
from jax.experimental import pallas as pl
from jax.experimental.pallas import tpu as pltpu

H = W = 384
HW = H * W
NC, FC = 3, 96
NPTS = 8192
NUM_OVER = 3 * NPTS
STEP1 = int(NPTS * 0.75)
STEP2 = NPTS - STEP1
BIG = 1 << 30


@functools.cache
def _consts():
    perm = jax.random.permutation(jax.random.key(42), HW)
    flat = ((perm // W) * W + perm % W).astype(jnp.int32)
    cand_flat = flat[:NUM_OVER]
    cov_flat = flat[HW - STEP2:]
    rank_map = jnp.full((HW,), BIG, jnp.int32).at[cand_flat].set(
        jnp.arange(NUM_OVER, dtype=jnp.int32)).reshape(H, W)
    cov_mask = jnp.zeros((HW,), jnp.int32).at[cov_flat].set(1).reshape(H, W)
    WhT = jax.image.resize(jnp.eye(128, dtype=jnp.float32), (384, 128),
                           method='bilinear')
    return rank_map, cov_mask, WhT


def _mask_kernel(coarse_ref, whT_ref, wh_ref, rank_ref, cov_ref,
                 mask_ref, logit_ref):
    hi = jax.lax.Precision.HIGHEST
    ls = []
    for c in range(NC):
        t = jax.lax.dot_general(whT_ref[...], coarse_ref[c],
                                (((1,), (0,)), ((), ())), precision=hi)
        u = jax.lax.dot_general(t, wh_ref[...],
                                (((1,), (0,)), ((), ())), precision=hi)
        logit_ref[c] = u
        ls.append(u)
    l0, l1, l2 = ls
    m = jnp.maximum(jnp.maximum(l0, l1), l2)
    e0, e1, e2 = jnp.exp(l0 - m), jnp.exp(l1 - m), jnp.exp(l2 - m)
    s = (e0 + e1) + e2
    p0, p1, p2 = e0 / s, e1 / s, e2 / s
    top = jnp.maximum(jnp.maximum(p0, p1), p2)
    mid = jnp.maximum(jnp.minimum(p0, p1),
                      jnp.minimum(jnp.maximum(p0, p1), p2))
    unc = 1.0 - (top - mid)

    bits = jax.lax.bitcast_convert_type(unc, jnp.int32)
    rank = rank_ref[...]
    is_cand = rank < BIG
    ones = jnp.ones_like(bits)
    zeros = jnp.zeros_like(bits)

    def cnt(pred):
        return jnp.sum(jnp.where(pred, ones, zeros))

    def body(i, lohi):
        lo, hi_ = lohi
        mid_ = (lo + hi_) // 2
        c = cnt(is_cand & (bits >= mid_))
        ok = c >= STEP1
        return jnp.where(ok, mid_, lo), jnp.where(ok, hi_, mid_)

    lo, hi_ = jax.lax.fori_loop(0, 31, body, (jnp.int32(0),
                                              jnp.int32(0x3F800001)))
    tstar = lo
    count_gt = cnt(is_cand & (bits > tstar))
    m_need = STEP1 - count_gt
    is_tie = is_cand & (bits == tstar)

    def body2(i, lohi):
        lo2, hi2 = lohi
        mid_ = (lo2 + hi2) // 2
        c = cnt(is_tie & (rank <= mid_))
        ok = c >= m_need
        return jnp.where(ok, lo2, mid_ + 1), jnp.where(ok, mid_, hi2)

    lo2, _ = jax.lax.fori_loop(0, 15, body2,
                               (jnp.int32(0), jnp.int32(NUM_OVER - 1)))
    sel = is_cand & (bits > tstar)
    sel = sel | (is_tie & (m_need > 0) & (rank <= lo2))
    mask_ref[...] = jnp.where(sel | (cov_ref[...] == 1), 1, 0)


def _coord(o, in_sz):
    f = jnp.clip((o.astype(jnp.float32) + 0.5) * (in_sz / 384.0) - 0.5,
                 0.0, in_sz - 1.0)
    i0 = jnp.minimum(f.astype(jnp.int32), in_sz - 2)
    return i0, f - i0.astype(jnp.float32)


def _prelu(x, a):
    return jnp.where(x > 0, x, a * x)


def kernel(fine_grained, coarse_pre, conv1_w, conv2_w, conv3_w, prelu_w):
    B = coarse_pre.shape[0]
    rank_map, cov_mask, WhT = _consts()
    Wh = WhT.T

    mask, logits = jax.vmap(
        lambda cb: pl.pallas_call(
            _mask_kernel,
            out_shape=(jax.ShapeDtypeStruct((H, W), jnp.int32),
                       jax.ShapeDtypeStruct((NC, H, W), jnp.float32)),
        )(cb, WhT, Wh, rank_map, cov_mask)
    )(coarse_pre)

    pos = jax.vmap(
        lambda mb: jnp.nonzero(mb.reshape(-1), size=NPTS)[0].astype(jnp.int32)
    )(mask)
    h = pos // W
    w = pos % W
    y0, dy = _coord(h, 192)
    x0, dx = _coord(w, 192)

    def finterp(fb, y0, x0, dy, dx):
        g = lambda yy, xx: fb[:, yy, xx]
        return (g(y0, x0) * (1 - dy) * (1 - dx)
                + g(y0, x0 + 1) * (1 - dy) * dx
                + g(y0 + 1, x0) * dy * (1 - dx)
                + g(y0 + 1, x0 + 1) * dy * dx).T

    sf = jax.vmap(finterp)(fine_grained, y0, x0, dy, dx)
    sc = jax.vmap(
        lambda lb, p: lb.reshape(NC, HW)[:, p].T)(logits, pos)

    x = jnp.concatenate([sf.reshape(-1), sc.reshape(-1)]).reshape(
        B, NPTS, FC + NC)
    last3 = x[..., -3:]
    l1 = _prelu(jnp.einsum('bnc,oc->bno', x, conv1_w), prelu_w)
    l2 = _prelu(jnp.einsum('bnc,oc->bno',
                           jnp.concatenate([l1, last3], -1), conv2_w),
                prelu_w)
    out = jnp.einsum('bnc,oc->bno',
                     jnp.concatenate([l2, last3], -1), conv3_w)
    return jnp.transpose(out, (0, 2, 1)), mask


# speedup vs baseline: 16.1308x; 1.1507x over previous
import functools

import jax
import jax.numpy as jnp
from jax import lax
from jax.experimental import pallas as pl
from jax.experimental.pallas import tpu as pltpu
from jax.experimental.pallas import tpu_sc as plsc

H = W = 384
HW = H * W
NC, FC = 3, 96
NPTS = 8192
NUM_OVER = 3 * NPTS
STEP1 = int(NPTS * 0.75)
STEP2 = NPTS - STEP1
BIG = 1 << 30
FH = FW = 192
CH = CW = 128
NSUB = 16
CHUNK = HW // NSUB
PCAP = 2048


@functools.cache
def _consts():
    perm = jax.random.permutation(jax.random.key(42), HW)
    flat = ((perm // W) * W + perm % W).astype(jnp.int32)
    cand_flat = flat[:NUM_OVER]
    cov_flat = flat[HW - STEP2:]
    rank_map = jnp.full((HW,), BIG, jnp.int32).at[cand_flat].set(
        jnp.arange(NUM_OVER, dtype=jnp.int32)).reshape(H, W)
    cov_mask = jnp.zeros((HW,), jnp.int32).at[cov_flat].set(1).reshape(H, W)
    WhT = jax.image.resize(jnp.eye(128, dtype=jnp.float32), (384, 128),
                           method='bilinear')
    return rank_map, cov_mask, WhT


def _mask_kernel(coarse_ref, whT_ref, wh_ref, rank_ref, cov_ref, mask_ref):
    hi = jax.lax.Precision.HIGHEST
    ls = []
    for c in range(NC):
        t = jax.lax.dot_general(whT_ref[...], coarse_ref[c],
                                (((1,), (0,)), ((), ())), precision=hi)
        u = jax.lax.dot_general(t, wh_ref[...],
                                (((1,), (0,)), ((), ())), precision=hi)
        ls.append(u)
    l0, l1, l2 = ls
    m = jnp.maximum(jnp.maximum(l0, l1), l2)
    e0, e1, e2 = jnp.exp(l0 - m), jnp.exp(l1 - m), jnp.exp(l2 - m)
    s = (e0 + e1) + e2
    p0, p1, p2 = e0 / s, e1 / s, e2 / s
    top = jnp.maximum(jnp.maximum(p0, p1), p2)
    mid = jnp.maximum(jnp.minimum(p0, p1),
                      jnp.minimum(jnp.maximum(p0, p1), p2))
    unc = 1.0 - (top - mid)

    bits = jax.lax.bitcast_convert_type(unc, jnp.int32)
    rank = rank_ref[...]
    is_cand = rank < BIG
    ones = jnp.ones_like(bits)
    zeros = jnp.zeros_like(bits)

    def cnt(pred):
        return jnp.sum(jnp.where(pred, ones, zeros))

    def body(i, lohi):
        lo, hi_ = lohi
        mid_ = (lo + hi_) // 2
        ok = cnt(is_cand & (bits >= mid_)) >= STEP1
        return jnp.where(ok, mid_, lo), jnp.where(ok, hi_, mid_)

    lo, _ = lax.fori_loop(0, 31, body, (jnp.int32(0), jnp.int32(0x3F800001)))
    tstar = lo
    m_need = STEP1 - cnt(is_cand & (bits > tstar))
    is_tie = is_cand & (bits == tstar)

    def body2(i, lohi):
        lo2, hi2 = lohi
        mid_ = (lo2 + hi2) // 2
        ok = cnt(is_tie & (rank <= mid_)) >= m_need
        return jnp.where(ok, lo2, mid_ + 1), jnp.where(ok, mid_, hi2)

    lo2, _ = lax.fori_loop(0, 15, body2,
                           (jnp.int32(0), jnp.int32(NUM_OVER - 1)))
    sel = is_cand & (bits > tstar)
    sel = sel | (is_tie & (m_need > 0) & (rank <= lo2))
    mask_ref[...] = jnp.where(sel | (cov_ref[...] == 1), 1, 0)


def _transpose_kernel(fine_ref, out_ref):
    out_ref[:, :FC] = fine_ref[0].T
    out_ref[:, FC:] = jnp.zeros((512, 128 - FC), jnp.float32)


def _sc_body(mask_hbm, fine_hbm, coarse_hbm, feat_hbm,
             counts_shared, mask_chunk, posbuf, coarse_v, counts_v,
             cnt_splat, idxa, idxb, g0, g1, outbuf, slotbuf,
             sem_g, sem_s):
    b = lax.axis_index("c")
    s = lax.axis_index("s")
    i16 = lambda v: jnp.int32(v)

    pltpu.sync_copy(mask_hbm.at[pl.ds(b * HW + s * CHUNK, CHUNK)],
                    mask_chunk)
    pltpu.sync_copy(coarse_hbm.at[pl.ds(b * (NC * CH * CW), NC * CH * CW)],
                    coarse_v)

    def c_body(i, cnt):
        lane = lax.iota(jnp.int32, 16)
        vec = mask_chunk[pl.ds(i * 16, 16)]
        posv = s * CHUNK + i * 16 + lane
        m = vec == 1
        p = jnp.where(m, 1, 0)
        dn = lax.GatherDimensionNumbers(offset_dims=(),
                                        collapsed_slice_dims=(0,),
                                        start_index_map=(0,))
        for k in (1, 2, 4, 8):
            sh = lax.gather(p, jnp.maximum(lane - k, 0)[:, None], dn,
                            slice_sizes=(1,),
                            mode=lax.GatherScatterMode.PROMISE_IN_BOUNDS)
            p = p + jnp.where(lane >= k, sh, 0)
        dest = jnp.where(m, cnt + p - 1, jnp.int32(PCAP + 8))
        plsc.store_scatter(posbuf, [dest], posv)
        return cnt + plsc.all_reduce_population_count(m)[0]

    cnt = lax.fori_loop(0, CHUNK // 16, c_body, i16(0))

    cnt_splat[...] = jnp.full((16,), cnt, jnp.int32)
    pltpu.sync_copy(cnt_splat, counts_shared.at[s])
    plsc.subcore_barrier()
    pltpu.sync_copy(counts_shared, counts_v)
    lane = lax.iota(jnp.int32, 16)
    allcnt = plsc.load_gather(counts_v, [lane, jnp.zeros((16,), jnp.int32)])
    off = i16(0)
    for j in range(NSUB):
        off = off + jnp.where(j < s, allcnt[j], 0)
    base = b * NPTS + off
    dump = jnp.int32(2 * NPTS)

    half = jnp.float32(0.5)

    def blk(i, _):
        lane = lax.iota(jnp.int32, 16)
        valid = (i * 16 + lane) < cnt
        pvec = jnp.where(valid, posbuf[pl.ds(i * 16, 16)], 0)
        h = pvec // W
        w = pvec % W
        hf = h.astype(jnp.float32)
        wf = w.astype(jnp.float32)
        yf = jnp.clip(hf * half - 0.25, 0.0, 191.0)
        xf = jnp.clip(wf * half - 0.25, 0.0, 191.0)
        y0 = jnp.minimum(yf.astype(jnp.int32), 190)
        x0 = jnp.minimum(xf.astype(jnp.int32), 190)
        dy = yf - y0.astype(jnp.float32)
        dx = xf - x0.astype(jnp.float32)
        cs = jnp.float32(128.0 / 384.0)
        ycf = jnp.clip((hf + half) * cs - half, 0.0, 127.0)
        xcf = jnp.clip((wf + half) * cs - half, 0.0, 127.0)
        yc0 = jnp.minimum(ycf.astype(jnp.int32), 126)
        xc0 = jnp.minimum(xcf.astype(jnp.int32), 126)
        dyc = ycf - yc0.astype(jnp.float32)
        dxc = xcf - xc0.astype(jnp.float32)

        r0 = b * (FH * FW) + y0 * FW + x0
        idxa[pl.ds(0, 16)] = r0
        idxa[pl.ds(16, 16)] = r0 + 1
        idxb[pl.ds(0, 16)] = r0 + FW
        idxb[pl.ds(16, 16)] = r0 + FW + 1
        cp0 = pltpu.async_copy(fine_hbm.at[idxa], g0, sem_g)
        cp1 = pltpu.async_copy(fine_hbm.at[idxb], g1, sem_g)

        cbase = yc0 * CW + xc0
        w00 = (1.0 - dyc) * (1.0 - dxc)
        w01 = (1.0 - dyc) * dxc
        w10 = dyc * (1.0 - dxc)
        w11 = dyc * dxc
        for c in range(NC):
            o = cbase + c * (CH * CW)
            v = (plsc.load_gather(coarse_v, [o]) * w00
                 + plsc.load_gather(coarse_v, [o + 1]) * w01
                 + plsc.load_gather(coarse_v, [o + CW]) * w10
                 + plsc.load_gather(coarse_v, [o + CW + 1]) * w11)
            plsc.store_scatter(outbuf,
                               [lane, jnp.full((16,), FC + c, jnp.int32)], v)

        cp0.wait()
        cp1.wait()
        for j in range(16):
            dyj = dy[j]
            dxj = dx[j]
            f00 = (1.0 - dyj) * (1.0 - dxj)
            f01 = (1.0 - dyj) * dxj
            f10 = dyj * (1.0 - dxj)
            f11 = dyj * dxj
            for v in range(FC // 16):
                sl = pl.ds(v * 16, 16)
                outbuf[j, sl] = (g0[j, sl] * f00 + g0[j + 16, sl] * f01
                                 + g1[j, sl] * f10 + g1[j + 16, sl] * f11)

        slotbuf[...] = jnp.where(valid, base + i * 16 + lane, dump)
        pltpu.async_copy(outbuf, feat_hbm.at[slotbuf], sem_s).wait()
        return 0

    nblk = (cnt + 15) // 16
    lax.fori_loop(0, nblk, blk, 0)


def _sc_gather(mask_flat, fine_cl, coarse_flat):
    mesh = plsc.VectorSubcoreMesh(core_axis_name="c", subcore_axis_name="s")
    f = pl.kernel(
        _sc_body,
        out_type=jax.ShapeDtypeStruct((2 * NPTS + 16, 128), jnp.float32),
        mesh=mesh,
        compiler_params=pltpu.CompilerParams(needs_layout_passes=False),
        scratch_types=[
            pltpu.VMEM_SHARED((NSUB, 16), jnp.int32),
            pltpu.VMEM((CHUNK,), jnp.int32),
            pltpu.VMEM((PCAP + 16,), jnp.int32),
            pltpu.VMEM((NC * CH * CW,), jnp.float32),
            pltpu.VMEM((NSUB, 16), jnp.int32),
            pltpu.VMEM((16,), jnp.int32),
            pltpu.VMEM((32,), jnp.int32),
            pltpu.VMEM((32,), jnp.int32),
            pltpu.VMEM((32, 128), jnp.float32),
            pltpu.VMEM((32, 128), jnp.float32),
            pltpu.VMEM((16, 128), jnp.float32),
            pltpu.VMEM((16,), jnp.int32),
            pltpu.SemaphoreType.DMA,
            pltpu.SemaphoreType.DMA,
        ],
    )
    return f(mask_flat, fine_cl, coarse_flat)


def _mlp_kernel(x_ref, w1_ref, w2a_ref, w2b_ref, w3a_ref, w3b_ref, a_ref,
                out_ref):
    a = a_ref[0, 0]
    x = x_ref[...]
    x3 = x[:, 96:99]
    h1 = jnp.dot(x, w1_ref[...])
    h1 = jnp.where(h1 > 0, h1, a * h1)
    h2 = jnp.dot(h1, w2a_ref[...]) + jnp.dot(x3, w2b_ref[...])
    h2 = jnp.where(h2 > 0, h2, a * h2)
    out_ref[...] = jnp.dot(h2, w3a_ref[...]) + jnp.dot(x3, w3b_ref[...])


def _mlp(x, conv1_w, conv2_w, conv3_w, prelu_w):
    n = x.shape[0]
    blk = 2048
    w1t = conv1_w.T
    w2at = conv2_w[:, :256].T
    w2bt = conv2_w[:, 256:].T
    w3at = conv3_w[:, :128].T
    w3bt = conv3_w[:, 128:].T
    a = prelu_w.reshape(1, 1)
    full = lambda shp: pl.BlockSpec(shp, lambda i: (0, 0))
    return pl.pallas_call(
        _mlp_kernel,
        grid=(n // blk,),
        in_specs=[pl.BlockSpec((blk, 99), lambda i: (i, 0)),
                  full((99, 256)), full((256, 128)), full((3, 128)),
                  full((128, 3)), full((3, 3)), full((1, 1))],
        out_specs=pl.BlockSpec((blk, 3), lambda i: (i, 0)),
        out_shape=jax.ShapeDtypeStruct((n, 3), jnp.float32),
    )(x, w1t, w2at, w2bt, w3at, w3bt, a)


def kernel(fine_grained, coarse_pre, conv1_w, conv2_w, conv3_w, prelu_w):
    B = coarse_pre.shape[0]
    rank_map, cov_mask, WhT = _consts()
    Wh = WhT.T

    mask = jax.vmap(
        lambda cb: pl.pallas_call(
            _mask_kernel,
            out_shape=jax.ShapeDtypeStruct((H, W), jnp.int32),
        )(cb, WhT, Wh, rank_map, cov_mask)
    )(coarse_pre)

    fine_cl = pl.pallas_call(
        _transpose_kernel,
        grid=(B, FH * FW // 512),
        in_specs=[pl.BlockSpec((1, FC, 512), lambda b, j: (b, 0, j))],
        out_specs=pl.BlockSpec((512, 128),
                               lambda b, j: (b * (FH * FW // 512) + j, 0)),
        out_shape=jax.ShapeDtypeStruct((B * FH * FW, 128), jnp.float32),
    )(fine_grained.reshape(B, FC, FH * FW))

    feat = _sc_gather(mask.reshape(B * HW),
                      fine_cl,
                      coarse_pre.reshape(B * NC * CH * CW))

    sf = feat[:B * NPTS, :FC]
    sc = feat[:B * NPTS, FC:FC + NC]
    x = jnp.concatenate([sf.reshape(-1), sc.reshape(-1)]).reshape(
        B * NPTS, FC + NC)
    out = _mlp(x, conv1_w, conv2_w, conv3_w, prelu_w)
    return out.reshape(B, NPTS, NC).transpose(0, 2, 1), mask


# speedup vs baseline: 36.1424x; 2.2406x over previous
import functools

import jax
import jax.numpy as jnp
from jax import lax
from jax.experimental import pallas as pl
from jax.experimental.pallas import tpu as pltpu
from jax.experimental.pallas import tpu_sc as plsc

H = W = 384
HW = H * W
NC, FC = 3, 96
NPTS = 8192
NUM_OVER = 3 * NPTS
STEP1 = int(NPTS * 0.75)
STEP2 = NPTS - STEP1
BIG = 1 << 30
FH = FW = 192
CH = CW = 128
NSUB = 16
CHUNK = HW // NSUB
PCAP = 2048


@functools.cache
def _consts():
    import numpy as np
    with jax.ensure_compile_time_eval():
        perm = np.asarray(jax.random.permutation(jax.random.key(42), HW))
        WhT = np.asarray(jax.image.resize(jnp.eye(128, dtype=jnp.float32),
                                          (384, 128), method='bilinear'))
    flat = ((perm // W) * W + perm % W).astype(np.int32)
    cand_flat = flat[:NUM_OVER]
    cov_flat = flat[HW - STEP2:]
    rank_map = np.full((HW,), BIG, np.int32)
    rank_map[cand_flat] = np.arange(NUM_OVER, dtype=np.int32)
    cov_mask = np.zeros((HW,), np.int32)
    cov_mask[cov_flat] = 1
    return rank_map.reshape(H, W), cov_mask.reshape(H, W), WhT


def _mask_kernel(coarse_ref, whT_ref, wh_ref, rank_ref, cov_ref, mask_ref):
    hi = jax.lax.Precision.HIGHEST
    ls = []
    for c in range(NC):
        t = jax.lax.dot_general(whT_ref[...], coarse_ref[c],
                                (((1,), (0,)), ((), ())), precision=hi)
        u = jax.lax.dot_general(t, wh_ref[...],
                                (((1,), (0,)), ((), ())), precision=hi)
        ls.append(u)
    l0, l1, l2 = ls
    m = jnp.maximum(jnp.maximum(l0, l1), l2)
    e0, e1, e2 = jnp.exp(l0 - m), jnp.exp(l1 - m), jnp.exp(l2 - m)
    s = (e0 + e1) + e2
    p0, p1, p2 = e0 / s, e1 / s, e2 / s
    top = jnp.maximum(jnp.maximum(p0, p1), p2)
    mid = jnp.maximum(jnp.minimum(p0, p1),
                      jnp.minimum(jnp.maximum(p0, p1), p2))
    unc = 1.0 - (top - mid)

    bits = jax.lax.bitcast_convert_type(unc, jnp.int32)
    rank = rank_ref[...]
    is_cand = rank < BIG
    ones = jnp.ones_like(bits)
    zeros = jnp.zeros_like(bits)

    def cnt(pred):
        return jnp.sum(jnp.where(pred, ones, zeros))

    def body(i, lohi):
        lo, hi_ = lohi
        mid_ = (lo + hi_) // 2
        ok = cnt(is_cand & (bits >= mid_)) >= STEP1
        return jnp.where(ok, mid_, lo), jnp.where(ok, hi_, mid_)

    lo, _ = lax.fori_loop(0, 31, body, (jnp.int32(0), jnp.int32(0x3F800001)))
    tstar = lo
    m_need = STEP1 - cnt(is_cand & (bits > tstar))
    is_tie = is_cand & (bits == tstar)

    def body2(i, lohi):
        lo2, hi2 = lohi
        mid_ = (lo2 + hi2) // 2
        ok = cnt(is_tie & (rank <= mid_)) >= m_need
        return jnp.where(ok, lo2, mid_ + 1), jnp.where(ok, mid_, hi2)

    lo2, _ = lax.fori_loop(0, 15, body2,
                           (jnp.int32(0), jnp.int32(NUM_OVER - 1)))
    sel = is_cand & (bits > tstar)
    sel = sel | (is_tie & (m_need > 0) & (rank <= lo2))
    mask_ref[...] = jnp.where(sel | (cov_ref[...] == 1), 1, 0)


def _transpose_kernel(fine_ref, out_ref):
    out_ref[:, :FC] = fine_ref[0].T
    out_ref[:, FC:] = jnp.zeros((512, 128 - FC), jnp.float32)


def _sc_body(mask_hbm, fine_hbm, coarse_hbm, feat_hbm,
             counts_shared, mask_chunk, posbuf, coarse_v, counts_v,
             cnt_splat, idxa, idxb, g0, g1, outbuf, slotbuf,
             sem_g, sem_s):
    b = lax.axis_index("c")
    s = lax.axis_index("s")
    i16 = lambda v: jnp.int32(v)

    pltpu.sync_copy(mask_hbm.at[pl.ds(b * HW + s * CHUNK, CHUNK)],
                    mask_chunk)
    pltpu.sync_copy(coarse_hbm.at[pl.ds(b * (NC * CH * CW), NC * CH * CW)],
                    coarse_v)

    def c_body(i, cnt):
        lane = lax.iota(jnp.int32, 16)
        vec = mask_chunk[pl.ds(i * 16, 16)]
        posv = s * CHUNK + i * 16 + lane
        m = vec == 1
        p = jnp.where(m, 1, 0)
        dn = lax.GatherDimensionNumbers(offset_dims=(),
                                        collapsed_slice_dims=(0,),
                                        start_index_map=(0,))
        for k in (1, 2, 4, 8):
            sh = lax.gather(p, jnp.maximum(lane - k, 0)[:, None], dn,
                            slice_sizes=(1,),
                            mode=lax.GatherScatterMode.PROMISE_IN_BOUNDS)
            p = p + jnp.where(lane >= k, sh, 0)
        dest = jnp.where(m, cnt + p - 1, jnp.int32(PCAP + 8))
        plsc.store_scatter(posbuf, [dest], posv)
        return cnt + plsc.all_reduce_population_count(m)[0]

    cnt = lax.fori_loop(0, CHUNK // 16, c_body, i16(0))

    cnt_splat[...] = jnp.full((16,), cnt, jnp.int32)
    pltpu.sync_copy(cnt_splat, counts_shared.at[s])
    plsc.subcore_barrier()
    pltpu.sync_copy(counts_shared, counts_v)
    lane = lax.iota(jnp.int32, 16)
    allcnt = plsc.load_gather(counts_v, [lane, jnp.zeros((16,), jnp.int32)])
    off = i16(0)
    for j in range(NSUB):
        off = off + jnp.where(j < s, allcnt[j], 0)
    base = b * NPTS + off
    dump = jnp.int32(2 * NPTS)

    half = jnp.float32(0.5)

    def blk(i, _):
        lane = lax.iota(jnp.int32, 16)
        valid = (i * 16 + lane) < cnt
        pvec = jnp.where(valid, posbuf[pl.ds(i * 16, 16)], 0)
        h = pvec // W
        w = pvec % W
        hf = h.astype(jnp.float32)
        wf = w.astype(jnp.float32)
        yf = jnp.clip(hf * half - 0.25, 0.0, 191.0)
        xf = jnp.clip(wf * half - 0.25, 0.0, 191.0)
        y0 = jnp.minimum(yf.astype(jnp.int32), 190)
        x0 = jnp.minimum(xf.astype(jnp.int32), 190)
        dy = yf - y0.astype(jnp.float32)
        dx = xf - x0.astype(jnp.float32)
        cs = jnp.float32(128.0 / 384.0)
        ycf = jnp.clip((hf + half) * cs - half, 0.0, 127.0)
        xcf = jnp.clip((wf + half) * cs - half, 0.0, 127.0)
        yc0 = jnp.minimum(ycf.astype(jnp.int32), 126)
        xc0 = jnp.minimum(xcf.astype(jnp.int32), 126)
        dyc = ycf - yc0.astype(jnp.float32)
        dxc = xcf - xc0.astype(jnp.float32)

        r0 = b * (FH * FW) + y0 * FW + x0
        idxa[pl.ds(0, 16)] = r0
        idxa[pl.ds(16, 16)] = r0 + 1
        idxb[pl.ds(0, 16)] = r0 + FW
        idxb[pl.ds(16, 16)] = r0 + FW + 1
        cp0 = pltpu.async_copy(fine_hbm.at[idxa], g0, sem_g)
        cp1 = pltpu.async_copy(fine_hbm.at[idxb], g1, sem_g)

        cbase = yc0 * CW + xc0
        w00 = (1.0 - dyc) * (1.0 - dxc)
        w01 = (1.0 - dyc) * dxc
        w10 = dyc * (1.0 - dxc)
        w11 = dyc * dxc
        for c in range(NC):
            o = cbase + c * (CH * CW)
            v = (plsc.load_gather(coarse_v, [o]) * w00
                 + plsc.load_gather(coarse_v, [o + 1]) * w01
                 + plsc.load_gather(coarse_v, [o + CW]) * w10
                 + plsc.load_gather(coarse_v, [o + CW + 1]) * w11)
            plsc.store_scatter(outbuf,
                               [lane, jnp.full((16,), FC + c, jnp.int32)], v)

        cp0.wait()
        cp1.wait()
        for j in range(16):
            dyj = dy[j]
            dxj = dx[j]
            f00 = (1.0 - dyj) * (1.0 - dxj)
            f01 = (1.0 - dyj) * dxj
            f10 = dyj * (1.0 - dxj)
            f11 = dyj * dxj
            for v in range(FC // 16):
                sl = pl.ds(v * 16, 16)
                outbuf[j, sl] = (g0[j, sl] * f00 + g0[j + 16, sl] * f01
                                 + g1[j, sl] * f10 + g1[j + 16, sl] * f11)

        slotbuf[...] = jnp.where(valid, base + i * 16 + lane, dump)
        pltpu.async_copy(outbuf, feat_hbm.at[slotbuf], sem_s).wait()
        return 0

    nblk = (cnt + 15) // 16
    lax.fori_loop(0, nblk, blk, 0)


def _sc_gather(mask_flat, fine_cl, coarse_flat):
    mesh = plsc.VectorSubcoreMesh(core_axis_name="c", subcore_axis_name="s")
    f = pl.kernel(
        _sc_body,
        out_type=jax.ShapeDtypeStruct((2 * NPTS + 16, 128), jnp.float32),
        mesh=mesh,
        compiler_params=pltpu.CompilerParams(needs_layout_passes=False),
        scratch_types=[
            pltpu.VMEM_SHARED((NSUB, 16), jnp.int32),
            pltpu.VMEM((CHUNK,), jnp.int32),
            pltpu.VMEM((PCAP + 16,), jnp.int32),
            pltpu.VMEM((NC * CH * CW,), jnp.float32),
            pltpu.VMEM((NSUB, 16), jnp.int32),
            pltpu.VMEM((16,), jnp.int32),
            pltpu.VMEM((32,), jnp.int32),
            pltpu.VMEM((32,), jnp.int32),
            pltpu.VMEM((32, 128), jnp.float32),
            pltpu.VMEM((32, 128), jnp.float32),
            pltpu.VMEM((16, 128), jnp.float32),
            pltpu.VMEM((16,), jnp.int32),
            pltpu.SemaphoreType.DMA,
            pltpu.SemaphoreType.DMA,
        ],
    )
    return f(mask_flat, fine_cl, coarse_flat)


def _mlp_kernel(x_ref, w1_ref, w2a_ref, w2b_ref, w3a_ref, w3b_ref, a_ref,
                out_ref):
    a = a_ref[0, 0]
    x = x_ref[...]
    x3 = x[:, 96:99]
    h1 = jnp.dot(x, w1_ref[...])
    h1 = jnp.where(h1 > 0, h1, a * h1)
    h2 = jnp.dot(h1, w2a_ref[...]) + jnp.dot(x3, w2b_ref[...])
    h2 = jnp.where(h2 > 0, h2, a * h2)
    out_ref[...] = jnp.dot(h2, w3a_ref[...]) + jnp.dot(x3, w3b_ref[...])


def _mlp(x, conv1_w, conv2_w, conv3_w, prelu_w):
    n = x.shape[0]
    blk = 2048
    w1t = conv1_w.T
    w2at = conv2_w[:, :256].T
    w2bt = conv2_w[:, 256:].T
    w3at = conv3_w[:, :128].T
    w3bt = conv3_w[:, 128:].T
    a = prelu_w.reshape(1, 1)
    full = lambda shp: pl.BlockSpec(shp, lambda i: (0, 0))
    return pl.pallas_call(
        _mlp_kernel,
        grid=(n // blk,),
        in_specs=[pl.BlockSpec((blk, 99), lambda i: (i, 0)),
                  full((99, 256)), full((256, 128)), full((3, 128)),
                  full((128, 3)), full((3, 3)), full((1, 1))],
        out_specs=pl.BlockSpec((blk, 3), lambda i: (i, 0)),
        out_shape=jax.ShapeDtypeStruct((n, 3), jnp.float32),
    )(x, w1t, w2at, w2bt, w3at, w3bt, a)


def kernel(fine_grained, coarse_pre, conv1_w, conv2_w, conv3_w, prelu_w):
    B = coarse_pre.shape[0]
    rank_map, cov_mask, WhT = _consts()
    Wh = WhT.T

    mask = jax.vmap(
        lambda cb: pl.pallas_call(
            _mask_kernel,
            out_shape=jax.ShapeDtypeStruct((H, W), jnp.int32),
        )(cb, WhT, Wh, rank_map, cov_mask)
    )(coarse_pre)

    fine_cl = pl.pallas_call(
        _transpose_kernel,
        grid=(B, FH * FW // 512),
        in_specs=[pl.BlockSpec((1, FC, 512), lambda b, j: (b, 0, j))],
        out_specs=pl.BlockSpec((512, 128),
                               lambda b, j: (b * (FH * FW // 512) + j, 0)),
        out_shape=jax.ShapeDtypeStruct((B * FH * FW, 128), jnp.float32),
    )(fine_grained.reshape(B, FC, FH * FW))

    feat = _sc_gather(mask.reshape(B * HW),
                      fine_cl,
                      coarse_pre.reshape(B * NC * CH * CW))

    sf = feat[:B * NPTS, :FC]
    sc = feat[:B * NPTS, FC:FC + NC]
    x = jnp.concatenate([sf.reshape(-1), sc.reshape(-1)]).reshape(
        B * NPTS, FC + NC)
    out = _mlp(x, conv1_w, conv2_w, conv3_w, prelu_w)
    return out.reshape(B, NPTS, NC).transpose(0, 2, 1), mask


# speedup vs baseline: 43.6780x; 1.2085x over previous
import functools

import jax
import jax.numpy as jnp
from jax import lax
from jax.experimental import pallas as pl
from jax.experimental.pallas import tpu as pltpu
from jax.experimental.pallas import tpu_sc as plsc

H = W = 384
HW = H * W
NC, FC = 3, 96
NPTS = 8192
NUM_OVER = 3 * NPTS
STEP1 = int(NPTS * 0.75)
STEP2 = NPTS - STEP1
BIG = 1 << 30
FH = FW = 192
CH = CW = 128
NSUB = 16
CHUNK = HW // NSUB
PCAP = 2048


@functools.cache
def _consts():
    import numpy as np
    with jax.ensure_compile_time_eval():
        perm = np.asarray(jax.random.permutation(jax.random.key(42), HW))
        WhT = np.asarray(jax.image.resize(jnp.eye(128, dtype=jnp.float32),
                                          (384, 128), method='bilinear'))
    flat = ((perm // W) * W + perm % W).astype(np.int32)
    cand_flat = flat[:NUM_OVER]
    cov_flat = flat[HW - STEP2:]
    rank_map = np.full((HW,), BIG, np.int32)
    rank_map[cand_flat] = np.arange(NUM_OVER, dtype=np.int32)
    cov_mask = np.zeros((HW,), np.int32)
    cov_mask[cov_flat] = 1
    return rank_map.reshape(H, W), cov_mask.reshape(H, W), WhT


def _mask_kernel(coarse_ref, whT_ref, wh_ref, rank_ref, cov_ref, mask_ref):
    hi = jax.lax.Precision.HIGHEST
    ls = []
    for c in range(NC):
        t = jax.lax.dot_general(whT_ref[...], coarse_ref[c],
                                (((1,), (0,)), ((), ())), precision=hi)
        u = jax.lax.dot_general(t, wh_ref[...],
                                (((1,), (0,)), ((), ())), precision=hi)
        ls.append(u)
    l0, l1, l2 = ls
    m = jnp.maximum(jnp.maximum(l0, l1), l2)
    e0, e1, e2 = jnp.exp(l0 - m), jnp.exp(l1 - m), jnp.exp(l2 - m)
    s = (e0 + e1) + e2
    p0, p1, p2 = e0 / s, e1 / s, e2 / s
    top = jnp.maximum(jnp.maximum(p0, p1), p2)
    mid = jnp.maximum(jnp.minimum(p0, p1),
                      jnp.minimum(jnp.maximum(p0, p1), p2))
    unc = 1.0 - (top - mid)

    bits = jax.lax.bitcast_convert_type(unc, jnp.int32)
    rank = rank_ref[...]
    is_cand = rank < BIG
    ones = jnp.ones_like(bits)
    zeros = jnp.zeros_like(bits)

    def cnt(pred):
        return jnp.sum(jnp.where(pred, ones, zeros))

    def body(i, lohi):
        lo, hi_ = lohi
        mid_ = (lo + hi_) // 2
        ok = cnt(is_cand & (bits >= mid_)) >= STEP1
        return jnp.where(ok, mid_, lo), jnp.where(ok, hi_, mid_)

    lo, _ = lax.fori_loop(0, 31, body, (jnp.int32(0), jnp.int32(0x3F800001)))
    tstar = lo
    m_need = STEP1 - cnt(is_cand & (bits > tstar))
    is_tie = is_cand & (bits == tstar)

    def body2(i, lohi):
        lo2, hi2 = lohi
        mid_ = (lo2 + hi2) // 2
        ok = cnt(is_tie & (rank <= mid_)) >= m_need
        return jnp.where(ok, lo2, mid_ + 1), jnp.where(ok, mid_, hi2)

    lo2, _ = lax.fori_loop(0, 15, body2,
                           (jnp.int32(0), jnp.int32(NUM_OVER - 1)))
    sel = is_cand & (bits > tstar)
    sel = sel | (is_tie & (m_need > 0) & (rank <= lo2))
    mask_ref[...] = jnp.where(sel | (cov_ref[...] == 1), 1, 0)


def _transpose_kernel(fine_ref, eye_ref, out_ref):
    x = fine_ref[0].reshape(FC, 8 * FW)
    out_ref[:, :FC] = jax.lax.dot_general(
        x, eye_ref[...], (((0,), (0,)), ((), ())),
        precision=jax.lax.Precision.HIGHEST)
    out_ref[:, FC:] = jnp.zeros((8 * FW, 128 - FC), jnp.float32)


def _sc_body(mask_hbm, fine_hbm, coarse_hbm, feat_hbm,
             counts_shared, mask_chunk, posbuf, coarse_v, counts_v,
             cnt_splat, idxa, idxb, g0, g1, outbuf, slotbuf,
             sem_g, sem_s):
    b = lax.axis_index("c")
    s = lax.axis_index("s")
    i16 = lambda v: jnp.int32(v)

    pltpu.sync_copy(mask_hbm.at[pl.ds(b * HW + s * CHUNK, CHUNK)],
                    mask_chunk)
    pltpu.sync_copy(coarse_hbm.at[pl.ds(b * (NC * CH * CW), NC * CH * CW)],
                    coarse_v)

    def c_body(i, cnt):
        lane = lax.iota(jnp.int32, 16)
        vec = mask_chunk[pl.ds(i * 16, 16)]
        posv = s * CHUNK + i * 16 + lane
        m = vec == 1
        p = jnp.where(m, 1, 0)
        dn = lax.GatherDimensionNumbers(offset_dims=(),
                                        collapsed_slice_dims=(0,),
                                        start_index_map=(0,))
        for k in (1, 2, 4, 8):
            sh = lax.gather(p, jnp.maximum(lane - k, 0)[:, None], dn,
                            slice_sizes=(1,),
                            mode=lax.GatherScatterMode.PROMISE_IN_BOUNDS)
            p = p + jnp.where(lane >= k, sh, 0)
        dest = jnp.where(m, cnt + p - 1, jnp.int32(PCAP + 8))
        plsc.store_scatter(posbuf, [dest], posv)
        return cnt + plsc.all_reduce_population_count(m)[0]

    cnt = lax.fori_loop(0, CHUNK // 16, c_body, i16(0))

    cnt_splat[...] = jnp.full((16,), cnt, jnp.int32)
    pltpu.sync_copy(cnt_splat, counts_shared.at[s])
    plsc.subcore_barrier()
    pltpu.sync_copy(counts_shared, counts_v)
    lane = lax.iota(jnp.int32, 16)
    allcnt = plsc.load_gather(counts_v, [lane, jnp.zeros((16,), jnp.int32)])
    off = i16(0)
    for j in range(NSUB):
        off = off + jnp.where(j < s, allcnt[j], 0)
    base = b * NPTS + off
    dump = jnp.int32(2 * NPTS)

    half = jnp.float32(0.5)

    def blk(i, _):
        lane = lax.iota(jnp.int32, 16)
        valid = (i * 16 + lane) < cnt
        pvec = jnp.where(valid, posbuf[pl.ds(i * 16, 16)], 0)
        h = pvec // W
        w = pvec % W
        hf = h.astype(jnp.float32)
        wf = w.astype(jnp.float32)
        yf = jnp.clip(hf * half - 0.25, 0.0, 191.0)
        xf = jnp.clip(wf * half - 0.25, 0.0, 191.0)
        y0 = jnp.minimum(yf.astype(jnp.int32), 190)
        x0 = jnp.minimum(xf.astype(jnp.int32), 190)
        dy = yf - y0.astype(jnp.float32)
        dx = xf - x0.astype(jnp.float32)
        cs = jnp.float32(128.0 / 384.0)
        ycf = jnp.clip((hf + half) * cs - half, 0.0, 127.0)
        xcf = jnp.clip((wf + half) * cs - half, 0.0, 127.0)
        yc0 = jnp.minimum(ycf.astype(jnp.int32), 126)
        xc0 = jnp.minimum(xcf.astype(jnp.int32), 126)
        dyc = ycf - yc0.astype(jnp.float32)
        dxc = xcf - xc0.astype(jnp.float32)

        r0 = b * (FH * FW) + y0 * FW + x0
        idxa[pl.ds(0, 16)] = r0
        idxa[pl.ds(16, 16)] = r0 + 1
        idxb[pl.ds(0, 16)] = r0 + FW
        idxb[pl.ds(16, 16)] = r0 + FW + 1
        cp0 = pltpu.async_copy(fine_hbm.at[idxa], g0, sem_g)
        cp1 = pltpu.async_copy(fine_hbm.at[idxb], g1, sem_g)

        cbase = yc0 * CW + xc0
        w00 = (1.0 - dyc) * (1.0 - dxc)
        w01 = (1.0 - dyc) * dxc
        w10 = dyc * (1.0 - dxc)
        w11 = dyc * dxc
        for c in range(NC):
            o = cbase + c * (CH * CW)
            v = (plsc.load_gather(coarse_v, [o]) * w00
                 + plsc.load_gather(coarse_v, [o + 1]) * w01
                 + plsc.load_gather(coarse_v, [o + CW]) * w10
                 + plsc.load_gather(coarse_v, [o + CW + 1]) * w11)
            plsc.store_scatter(outbuf,
                               [lane, jnp.full((16,), FC + c, jnp.int32)], v)

        cp0.wait()
        cp1.wait()
        for j in range(16):
            dyj = dy[j]
            dxj = dx[j]
            f00 = (1.0 - dyj) * (1.0 - dxj)
            f01 = (1.0 - dyj) * dxj
            f10 = dyj * (1.0 - dxj)
            f11 = dyj * dxj
            for v in range(FC // 16):
                sl = pl.ds(v * 16, 16)
                outbuf[j, sl] = (g0[j, sl] * f00 + g0[j + 16, sl] * f01
                                 + g1[j, sl] * f10 + g1[j + 16, sl] * f11)

        slotbuf[...] = jnp.where(valid, base + i * 16 + lane, dump)
        pltpu.async_copy(outbuf, feat_hbm.at[slotbuf], sem_s).wait()
        return 0

    nblk = (cnt + 15) // 16
    lax.fori_loop(0, nblk, blk, 0)


def _sc_gather(mask_flat, fine_cl, coarse_flat):
    mesh = plsc.VectorSubcoreMesh(core_axis_name="c", subcore_axis_name="s")
    f = pl.kernel(
        _sc_body,
        out_type=jax.ShapeDtypeStruct((2 * NPTS + 16, 128), jnp.float32),
        mesh=mesh,
        compiler_params=pltpu.CompilerParams(needs_layout_passes=False),
        scratch_types=[
            pltpu.VMEM_SHARED((NSUB, 16), jnp.int32),
            pltpu.VMEM((CHUNK,), jnp.int32),
            pltpu.VMEM((PCAP + 16,), jnp.int32),
            pltpu.VMEM((NC * CH * CW,), jnp.float32),
            pltpu.VMEM((NSUB, 16), jnp.int32),
            pltpu.VMEM((16,), jnp.int32),
            pltpu.VMEM((32,), jnp.int32),
            pltpu.VMEM((32,), jnp.int32),
            pltpu.VMEM((32, 128), jnp.float32),
            pltpu.VMEM((32, 128), jnp.float32),
            pltpu.VMEM((16, 128), jnp.float32),
            pltpu.VMEM((16,), jnp.int32),
            pltpu.SemaphoreType.DMA,
            pltpu.SemaphoreType.DMA,
        ],
    )
    return f(mask_flat, fine_cl, coarse_flat)


def _mlp_kernel(x_ref, w1_ref, w2a_ref, w2b_ref, w3a_ref, w3b_ref, a_ref,
                out_ref):
    a = a_ref[0, 0]
    x = x_ref[...]
    x3 = x[:, 96:99]
    h1 = jnp.dot(x, w1_ref[...])
    h1 = jnp.where(h1 > 0, h1, a * h1)
    h2 = jnp.dot(h1, w2a_ref[...]) + jnp.dot(x3, w2b_ref[...])
    h2 = jnp.where(h2 > 0, h2, a * h2)
    out_ref[...] = jnp.dot(h2, w3a_ref[...]) + jnp.dot(x3, w3b_ref[...])


def _mlp(x, conv1_w, conv2_w, conv3_w, prelu_w):
    n = x.shape[0]
    blk = 2048
    w1t = conv1_w.T
    w2at = conv2_w[:, :256].T
    w2bt = conv2_w[:, 256:].T
    w3at = conv3_w[:, :128].T
    w3bt = conv3_w[:, 128:].T
    a = prelu_w.reshape(1, 1)
    full = lambda shp: pl.BlockSpec(shp, lambda i: (0, 0))
    return pl.pallas_call(
        _mlp_kernel,
        grid=(n // blk,),
        in_specs=[pl.BlockSpec((blk, 99), lambda i: (i, 0)),
                  full((99, 256)), full((256, 128)), full((3, 128)),
                  full((128, 3)), full((3, 3)), full((1, 1))],
        out_specs=pl.BlockSpec((blk, 3), lambda i: (i, 0)),
        out_shape=jax.ShapeDtypeStruct((n, 3), jnp.float32),
    )(x, w1t, w2at, w2bt, w3at, w3bt, a)


def kernel(fine_grained, coarse_pre, conv1_w, conv2_w, conv3_w, prelu_w):
    B = coarse_pre.shape[0]
    rank_map, cov_mask, WhT = _consts()
    Wh = WhT.T

    mask = jax.vmap(
        lambda cb: pl.pallas_call(
            _mask_kernel,
            out_shape=jax.ShapeDtypeStruct((H, W), jnp.int32),
        )(cb, WhT, Wh, rank_map, cov_mask)
    )(coarse_pre)

    nj = FH // 8
    fine_cl = pl.pallas_call(
        _transpose_kernel,
        grid=(B, nj),
        in_specs=[pl.BlockSpec((1, FC, 8, FW), lambda b, j: (b, 0, j, 0)),
                  pl.BlockSpec((FC, FC), lambda b, j: (0, 0))],
        out_specs=pl.BlockSpec((8 * FW, 128),
                               lambda b, j: (b * nj + j, 0)),
        out_shape=jax.ShapeDtypeStruct((B * FH * FW, 128), jnp.float32),
    )(fine_grained, jnp.eye(FC, dtype=jnp.float32))

    feat = _sc_gather(mask.reshape(B * HW),
                      fine_cl,
                      coarse_pre.reshape(B * NC * CH * CW))

    sf = feat[:B * NPTS, :FC]
    sc = feat[:B * NPTS, FC:FC + NC]
    x = jnp.concatenate([sf.reshape(-1), sc.reshape(-1)]).reshape(
        B * NPTS, FC + NC)
    out = _mlp(x, conv1_w, conv2_w, conv3_w, prelu_w)
    return out.reshape(B, NPTS, NC).transpose(0, 2, 1), mask


# speedup vs baseline: 47.6066x; 1.0899x over previous
import functools

import jax
import jax.numpy as jnp
from jax import lax
from jax.experimental import pallas as pl
from jax.experimental.pallas import tpu as pltpu
from jax.experimental.pallas import tpu_sc as plsc

H = W = 384
HW = H * W
NC, FC = 3, 96
NPTS = 8192
NUM_OVER = 3 * NPTS
STEP1 = int(NPTS * 0.75)
STEP2 = NPTS - STEP1
BIG = 1 << 30
FH = FW = 192
CH = CW = 128
NSUB = 16
CHUNK = HW // NSUB
PCAP = 2048


@functools.cache
def _consts():
    import numpy as np
    cpu = jax.local_devices(backend="cpu")[0]
    with jax.ensure_compile_time_eval(), jax.default_device(cpu):
        perm = np.asarray(jax.random.permutation(jax.random.key(42), HW))
        WhT = np.asarray(jax.image.resize(jnp.eye(128, dtype=jnp.float32),
                                          (384, 128), method='bilinear'))
    flat = ((perm // W) * W + perm % W).astype(np.int32)
    cand_flat = flat[:NUM_OVER]
    cov_flat = flat[HW - STEP2:]
    rank_map = np.full((HW,), BIG, np.int32)
    rank_map[cand_flat] = np.arange(NUM_OVER, dtype=np.int32)
    cov_mask = np.zeros((HW,), np.int32)
    cov_mask[cov_flat] = 1
    return rank_map.reshape(H, W), cov_mask.reshape(H, W), WhT


def _mask_kernel(coarse_ref, whT_ref, wh_ref, rank_ref, cov_ref, mask_ref):
    hi = jax.lax.Precision.HIGHEST
    ls = []
    for c in range(NC):
        t = jax.lax.dot_general(whT_ref[...], coarse_ref[c],
                                (((1,), (0,)), ((), ())), precision=hi)
        u = jax.lax.dot_general(t, wh_ref[...],
                                (((1,), (0,)), ((), ())), precision=hi)
        ls.append(u)
    l0, l1, l2 = ls
    m = jnp.maximum(jnp.maximum(l0, l1), l2)
    e0, e1, e2 = jnp.exp(l0 - m), jnp.exp(l1 - m), jnp.exp(l2 - m)
    s = (e0 + e1) + e2
    p0, p1, p2 = e0 / s, e1 / s, e2 / s
    top = jnp.maximum(jnp.maximum(p0, p1), p2)
    mid = jnp.maximum(jnp.minimum(p0, p1),
                      jnp.minimum(jnp.maximum(p0, p1), p2))
    unc = 1.0 - (top - mid)

    bits = jax.lax.bitcast_convert_type(unc, jnp.int32)
    rank = rank_ref[...]
    is_cand = rank < BIG
    ones = jnp.ones_like(bits)
    zeros = jnp.zeros_like(bits)

    def cnt(pred):
        return jnp.sum(jnp.where(pred, ones, zeros))

    def body(i, lohi):
        lo, hi_ = lohi
        mid_ = (lo + hi_) // 2
        ok = cnt(is_cand & (bits >= mid_)) >= STEP1
        return jnp.where(ok, mid_, lo), jnp.where(ok, hi_, mid_)

    lo, _ = lax.fori_loop(0, 31, body, (jnp.int32(0), jnp.int32(0x3F800001)))
    tstar = lo
    m_need = STEP1 - cnt(is_cand & (bits > tstar))
    is_tie = is_cand & (bits == tstar)

    def body2(i, lohi):
        lo2, hi2 = lohi
        mid_ = (lo2 + hi2) // 2
        ok = cnt(is_tie & (rank <= mid_)) >= m_need
        return jnp.where(ok, lo2, mid_ + 1), jnp.where(ok, mid_, hi2)

    lo2, _ = lax.fori_loop(0, 15, body2,
                           (jnp.int32(0), jnp.int32(NUM_OVER - 1)))
    sel = is_cand & (bits > tstar)
    sel = sel | (is_tie & (m_need > 0) & (rank <= lo2))
    mask_ref[...] = jnp.where(sel | (cov_ref[...] == 1), 1, 0)


def _transpose_kernel(fine_ref, eye_ref, out_ref):
    x = fine_ref[0].reshape(FC, 8 * FW)
    out_ref[:, :FC] = jax.lax.dot_general(
        x, eye_ref[...], (((0,), (0,)), ((), ())),
        precision=jax.lax.Precision.HIGHEST)
    out_ref[:, FC:] = jnp.zeros((8 * FW, 128 - FC), jnp.float32)


def _sc_body(mask_hbm, fine_hbm, coarse_hbm, feat_hbm,
             counts_shared, mask_chunk, posbuf, coarse_v, counts_v,
             cnt_splat, idxa0, idxb0, idxa1, idxb1, g00, g10, g01, g11,
             outb0, outb1, slot0, slot1, sg0, sg1, ss0, ss1):
    b = lax.axis_index("c")
    s = lax.axis_index("s")
    i16 = lambda v: jnp.int32(v)

    pltpu.sync_copy(mask_hbm.at[pl.ds(b * HW + s * CHUNK, CHUNK)],
                    mask_chunk)
    pltpu.sync_copy(coarse_hbm.at[pl.ds(b * (NC * CH * CW), NC * CH * CW)],
                    coarse_v)

    def c_body(i, cnt):
        lane = lax.iota(jnp.int32, 16)
        vec = mask_chunk[pl.ds(i * 16, 16)]
        posv = s * CHUNK + i * 16 + lane
        m = vec == 1
        p = jnp.where(m, 1, 0)
        dn = lax.GatherDimensionNumbers(offset_dims=(),
                                        collapsed_slice_dims=(0,),
                                        start_index_map=(0,))
        for k in (1, 2, 4, 8):
            sh = lax.gather(p, jnp.maximum(lane - k, 0)[:, None], dn,
                            slice_sizes=(1,),
                            mode=lax.GatherScatterMode.PROMISE_IN_BOUNDS)
            p = p + jnp.where(lane >= k, sh, 0)
        dest = jnp.where(m, cnt + p - 1, jnp.int32(PCAP + 8))
        plsc.store_scatter(posbuf, [dest], posv)
        return cnt + plsc.all_reduce_population_count(m)[0]

    cnt = lax.fori_loop(0, CHUNK // 16, c_body, i16(0))

    cnt_splat[...] = jnp.full((16,), cnt, jnp.int32)
    pltpu.sync_copy(cnt_splat, counts_shared.at[s])
    plsc.subcore_barrier()
    pltpu.sync_copy(counts_shared, counts_v)
    lane = lax.iota(jnp.int32, 16)
    allcnt = plsc.load_gather(counts_v, [lane, jnp.zeros((16,), jnp.int32)])
    off = i16(0)
    for j in range(NSUB):
        off = off + jnp.where(j < s, allcnt[j], 0)
    base = b * NPTS + off
    dump = jnp.int32(2 * NPTS)

    half = jnp.float32(0.5)
    nblk = (cnt + 15) // 16

    def prep(i):
        lane = lax.iota(jnp.int32, 16)
        valid = (i * 16 + lane) < cnt
        pvec = jnp.where(valid, posbuf[pl.ds(i * 16, 16)], 0)
        h = pvec // W
        w = pvec % W
        hf = h.astype(jnp.float32)
        wf = w.astype(jnp.float32)
        yf = jnp.clip(hf * half - 0.25, 0.0, 191.0)
        xf = jnp.clip(wf * half - 0.25, 0.0, 191.0)
        y0 = jnp.minimum(yf.astype(jnp.int32), 190)
        x0 = jnp.minimum(xf.astype(jnp.int32), 190)
        return lane, valid, hf, wf, y0, x0, yf, xf

    def fire(i, idxa, idxb, g0, g1, sg):
        _, _, _, _, y0, x0, _, _ = prep(i)
        r0 = b * (FH * FW) + y0 * FW + x0
        idxa[pl.ds(0, 16)] = r0
        idxa[pl.ds(16, 16)] = r0 + 1
        idxb[pl.ds(0, 16)] = r0 + FW
        idxb[pl.ds(16, 16)] = r0 + FW + 1
        pltpu.async_copy(fine_hbm.at[idxa], g0, sg)
        pltpu.async_copy(fine_hbm.at[idxb], g1, sg)

    def wait_g(idxa, idxb, g0, g1, sg):
        pltpu.make_async_copy(fine_hbm.at[idxa], g0, sg).wait()
        pltpu.make_async_copy(fine_hbm.at[idxb], g1, sg).wait()

    def wait_s(outbuf, slot, ss):
        pltpu.make_async_copy(outbuf, feat_hbm.at[slot], ss).wait()

    def blend(i, g0, g1, outbuf, slot, ss):
        lane, valid, hf, wf, y0, x0, yf, xf = prep(i)
        dy = yf - y0.astype(jnp.float32)
        dx = xf - x0.astype(jnp.float32)
        cs = jnp.float32(128.0 / 384.0)
        ycf = jnp.clip((hf + half) * cs - half, 0.0, 127.0)
        xcf = jnp.clip((wf + half) * cs - half, 0.0, 127.0)
        yc0 = jnp.minimum(ycf.astype(jnp.int32), 126)
        xc0 = jnp.minimum(xcf.astype(jnp.int32), 126)
        dyc = ycf - yc0.astype(jnp.float32)
        dxc = xcf - xc0.astype(jnp.float32)
        cbase = yc0 * CW + xc0
        w00 = (1.0 - dyc) * (1.0 - dxc)
        w01 = (1.0 - dyc) * dxc
        w10 = dyc * (1.0 - dxc)
        w11 = dyc * dxc
        for c in range(NC):
            o = cbase + c * (CH * CW)
            v = (plsc.load_gather(coarse_v, [o]) * w00
                 + plsc.load_gather(coarse_v, [o + 1]) * w01
                 + plsc.load_gather(coarse_v, [o + CW]) * w10
                 + plsc.load_gather(coarse_v, [o + CW + 1]) * w11)
            plsc.store_scatter(outbuf,
                               [lane, jnp.full((16,), FC + c, jnp.int32)], v)
        for j in range(16):
            dyj = dy[j]
            dxj = dx[j]
            f00 = (1.0 - dyj) * (1.0 - dxj)
            f01 = (1.0 - dyj) * dxj
            f10 = dyj * (1.0 - dxj)
            f11 = dyj * dxj
            for v in range(FC // 16):
                sl = pl.ds(v * 16, 16)
                outbuf[j, sl] = (g0[j, sl] * f00 + g0[j + 16, sl] * f01
                                 + g1[j, sl] * f10 + g1[j + 16, sl] * f11)
        slot[...] = jnp.where(valid, base + i * 16 + lane, dump)
        pltpu.async_copy(outbuf, feat_hbm.at[slot], ss)

    @pl.when(nblk > 0)
    def _():
        fire(0, idxa0, idxb0, g00, g10, sg0)

    def pair(k, _):
        i1 = 2 * k + 1

        @pl.when(i1 < nblk)
        def _():
            fire(i1, idxa1, idxb1, g01, g11, sg1)

        wait_g(idxa0, idxb0, g00, g10, sg0)

        @pl.when(k > 0)
        def _():
            wait_s(outb0, slot0, ss0)

        blend(2 * k, g00, g10, outb0, slot0, ss0)

        @pl.when(i1 < nblk)
        def _():
            @pl.when(i1 + 1 < nblk)
            def _():
                fire(i1 + 1, idxa0, idxb0, g00, g10, sg0)

            wait_g(idxa1, idxb1, g01, g11, sg1)

            @pl.when(k > 0)
            def _():
                wait_s(outb1, slot1, ss1)

            blend(i1, g01, g11, outb1, slot1, ss1)

        return 0

    lax.fori_loop(0, (nblk + 1) // 2, pair, 0)

    @pl.when(nblk >= 1)
    def _():
        wait_s(outb0, slot0, ss0)

    @pl.when(nblk >= 2)
    def _():
        wait_s(outb1, slot1, ss1)


def _sc_gather(mask_flat, fine_cl, coarse_flat):
    mesh = plsc.VectorSubcoreMesh(core_axis_name="c", subcore_axis_name="s")
    f = pl.kernel(
        _sc_body,
        out_type=jax.ShapeDtypeStruct((2 * NPTS + 16, 128), jnp.float32),
        mesh=mesh,
        compiler_params=pltpu.CompilerParams(needs_layout_passes=False),
        scratch_types=[
            pltpu.VMEM_SHARED((NSUB, 16), jnp.int32),
            pltpu.VMEM((CHUNK,), jnp.int32),
            pltpu.VMEM((PCAP + 16,), jnp.int32),
            pltpu.VMEM((NC * CH * CW,), jnp.float32),
            pltpu.VMEM((NSUB, 16), jnp.int32),
            pltpu.VMEM((16,), jnp.int32),
            pltpu.VMEM((32,), jnp.int32),
            pltpu.VMEM((32,), jnp.int32),
            pltpu.VMEM((32,), jnp.int32),
            pltpu.VMEM((32,), jnp.int32),
            pltpu.VMEM((32, 128), jnp.float32),
            pltpu.VMEM((32, 128), jnp.float32),
            pltpu.VMEM((32, 128), jnp.float32),
            pltpu.VMEM((32, 128), jnp.float32),
            pltpu.VMEM((16, 128), jnp.float32),
            pltpu.VMEM((16, 128), jnp.float32),
            pltpu.VMEM((16,), jnp.int32),
            pltpu.VMEM((16,), jnp.int32),
            pltpu.SemaphoreType.DMA,
            pltpu.SemaphoreType.DMA,
            pltpu.SemaphoreType.DMA,
            pltpu.SemaphoreType.DMA,
        ],
    )
    return f(mask_flat, fine_cl, coarse_flat)


def _mlp_kernel(x_ref, w1_ref, w2a_ref, w2b_ref, w3a_ref, w3b_ref, a_ref,
                out_ref):
    a = a_ref[0, 0]
    x = x_ref[...]
    x3 = x[:, 96:99]
    h1 = jnp.dot(x, w1_ref[...])
    h1 = jnp.where(h1 > 0, h1, a * h1)
    h2 = jnp.dot(h1, w2a_ref[...]) + jnp.dot(x3, w2b_ref[...])
    h2 = jnp.where(h2 > 0, h2, a * h2)
    out_ref[...] = jnp.dot(h2, w3a_ref[...]) + jnp.dot(x3, w3b_ref[...])


def _mlp(x, conv1_w, conv2_w, conv3_w, prelu_w):
    n = x.shape[0]
    blk = 2048
    w1t = conv1_w.T
    w2at = conv2_w[:, :256].T
    w2bt = conv2_w[:, 256:].T
    w3at = conv3_w[:, :128].T
    w3bt = conv3_w[:, 128:].T
    a = prelu_w.reshape(1, 1)
    full = lambda shp: pl.BlockSpec(shp, lambda i: (0, 0))
    return pl.pallas_call(
        _mlp_kernel,
        grid=(n // blk,),
        in_specs=[pl.BlockSpec((blk, 99), lambda i: (i, 0)),
                  full((99, 256)), full((256, 128)), full((3, 128)),
                  full((128, 3)), full((3, 3)), full((1, 1))],
        out_specs=pl.BlockSpec((blk, 3), lambda i: (i, 0)),
        out_shape=jax.ShapeDtypeStruct((n, 3), jnp.float32),
    )(x, w1t, w2at, w2bt, w3at, w3bt, a)


def kernel(fine_grained, coarse_pre, conv1_w, conv2_w, conv3_w, prelu_w):
    B = coarse_pre.shape[0]
    rank_map, cov_mask, WhT = _consts()
    Wh = WhT.T

    mask = jax.vmap(
        lambda cb: pl.pallas_call(
            _mask_kernel,
            out_shape=jax.ShapeDtypeStruct((H, W), jnp.int32),
        )(cb, WhT, Wh, rank_map, cov_mask)
    )(coarse_pre)

    nj = FH // 8
    fine_cl = pl.pallas_call(
        _transpose_kernel,
        grid=(B, nj),
        in_specs=[pl.BlockSpec((1, FC, 8, FW), lambda b, j: (b, 0, j, 0)),
                  pl.BlockSpec((FC, FC), lambda b, j: (0, 0))],
        out_specs=pl.BlockSpec((8 * FW, 128),
                               lambda b, j: (b * nj + j, 0)),
        out_shape=jax.ShapeDtypeStruct((B * FH * FW, 128), jnp.float32),
    )(fine_grained, jnp.eye(FC, dtype=jnp.float32))

    feat = _sc_gather(mask.reshape(B * HW),
                      fine_cl,
                      coarse_pre.reshape(B * NC * CH * CW))

    sf = feat[:B * NPTS, :FC]
    sc = feat[:B * NPTS, FC:FC + NC]
    x = jnp.concatenate([sf.reshape(-1), sc.reshape(-1)]).reshape(
        B * NPTS, FC + NC)
    out = _mlp(x, conv1_w, conv2_w, conv3_w, prelu_w)
    return out.reshape(B, NPTS, NC).transpose(0, 2, 1), mask
